# baseline (speedup 1.0000x reference)
import functools

import jax
import jax.numpy as jnp
from jax.experimental import pallas as pl
from jax.experimental.pallas import tpu as pltpu


def _gru_scan_body(T, x_ref, h0_ref, wi_ref, wh_ref, bi_ref, bh_ref, hf_ref,
                   gr_s, gz_s, gn_s):
    c = pl.program_id(0)

    @pl.when(c == 0)
    def _init():
        hf_ref[...] = h0_ref[...]

    x = x_ref[...]
    dn = (((2,), (0,)), ((), ()))
    f32 = jnp.float32
    gr_s[...] = jax.lax.dot_general(x, wi_ref[0], dn,
                                    preferred_element_type=f32) + bi_ref[0]
    gz_s[...] = jax.lax.dot_general(x, wi_ref[1], dn,
                                    preferred_element_type=f32) + bi_ref[1]
    gn_s[...] = jax.lax.dot_general(x, wi_ref[2], dn,
                                    preferred_element_type=f32) + bi_ref[2]

    whr = wh_ref[0]
    whz = wh_ref[1]
    whn = wh_ref[2]
    bhr = bh_ref[0]
    bhz = bh_ref[1]
    bhn = bh_ref[2]

    def step(t, h):
        ghr = jnp.dot(h, whr, preferred_element_type=f32) + bhr
        ghz = jnp.dot(h, whz, preferred_element_type=f32) + bhz
        ghn = jnp.dot(h, whn, preferred_element_type=f32) + bhn
        r = jax.nn.sigmoid(gr_s[t] + ghr)
        z = jax.nn.sigmoid(gz_s[t] + ghz)
        n = jnp.tanh(gn_s[t] + r * ghn)
        return (1.0 - z) * n + z * h

    h = jax.lax.fori_loop(0, T, step, hf_ref[...])
    hf_ref[...] = h


def _fwd_last_hidden(xs, h0, wi, wh, bi, bh, T):
    S, B, H = xs.shape
    full3 = lambda c: (0, 0, 0)
    return pl.pallas_call(
        functools.partial(_gru_scan_body, T),
        grid=(S // T,),
        in_specs=[
            pl.BlockSpec((T, B, H), lambda c: (c, 0, 0)),
            pl.BlockSpec((B, H), lambda c: (0, 0)),
            pl.BlockSpec((3, H, H), full3),
            pl.BlockSpec((3, H, H), full3),
            pl.BlockSpec((3, 1, H), full3),
            pl.BlockSpec((3, 1, H), full3),
        ],
        out_specs=pl.BlockSpec((B, H), lambda c: (0, 0)),
        out_shape=jax.ShapeDtypeStruct((B, H), jnp.float32),
        scratch_shapes=[
            pltpu.VMEM((T, B, H), jnp.float32),
            pltpu.VMEM((T, B, H), jnp.float32),
            pltpu.VMEM((T, B, H), jnp.float32),
        ],
    )(xs, h0, wi, wh, bi, bh)


def _tail_body(ha_ref, hf_ref, xa_ref, xf_ref,
               awb_ref, abib_ref, abhb_ref,
               fwb_ref, fbib_ref, fbhb_ref,
               p_ref, pb_ref, out_ref):
    f32 = jnp.float32

    def back_step(xt, wb_ref, bib_ref, bhb_ref):
        gr = jnp.dot(xt, wb_ref[0], preferred_element_type=f32) + bib_ref[0]
        gz = jnp.dot(xt, wb_ref[1], preferred_element_type=f32) + bib_ref[1]
        gn = jnp.dot(xt, wb_ref[2], preferred_element_type=f32) + bib_ref[2]
        r = jax.nn.sigmoid(gr + bhb_ref[0])
        z = jax.nn.sigmoid(gz + bhb_ref[1])
        n = jnp.tanh(gn + r * bhb_ref[2])
        return (1.0 - z) * n

    bA = back_step(xa_ref[...], awb_ref, abib_ref, abhb_ref)
    bF = back_step(xf_ref[...], fwb_ref, fbib_ref, fbhb_ref)
    acc = jnp.dot(ha_ref[...], p_ref[0], preferred_element_type=f32)
    acc = acc + jnp.dot(bA, p_ref[1], preferred_element_type=f32)
    acc = acc + jnp.dot(hf_ref[...], p_ref[2], preferred_element_type=f32)
    acc = acc + jnp.dot(bF, p_ref[3], preferred_element_type=f32)
    out_ref[...] = acc + pb_ref[...]


def _split_gates(W):
    H = W.shape[1]
    return jnp.swapaxes(W.reshape(3, H, H), 1, 2)


def kernel(atom_rep, frag_rep,
           atom_Wih_f, atom_Whh_f, atom_bih_f, atom_bhh_f,
           atom_Wih_b, atom_Whh_b, atom_bih_b, atom_bhh_b,
           frag_Wih_f, frag_Whh_f, frag_bih_f, frag_bhh_f,
           frag_Wih_b, frag_Whh_b, frag_bih_b, frag_bhh_b,
           proj_W, proj_b):
    B, S_atom, H = atom_rep.shape

    a_wi = _split_gates(atom_Wih_f)
    a_wh = _split_gates(atom_Whh_f)
    a_bi = atom_bih_f.reshape(3, 1, H)
    a_bh = atom_bhh_f.reshape(3, 1, H)

    NQ = 4
    SQ = S_atom // NQ
    hA = jnp.zeros((B, H), jnp.float32)
    for q in range(NQ):
        xa_q = jnp.swapaxes(
            jax.lax.slice_in_dim(atom_rep, q * SQ, (q + 1) * SQ, axis=1), 0, 1)
        hA = _fwd_last_hidden(xa_q, hA, a_wi, a_wh, a_bi, a_bh, 16)

    xf = jnp.swapaxes(frag_rep, 0, 1)
    hF = _fwd_last_hidden(xf, jnp.zeros((B, H), jnp.float32),
                          _split_gates(frag_Wih_f), _split_gates(frag_Whh_f),
                          frag_bih_f.reshape(3, 1, H),
                          frag_bhh_f.reshape(3, 1, H), 16)

    projT = proj_W.T.reshape(4, H, H)
    full3 = lambda: (0, 0, 0)
    full2 = lambda: (0, 0)
    out = pl.pallas_call(
        _tail_body,
        in_specs=[
            pl.BlockSpec((B, H), full2),
            pl.BlockSpec((B, H), full2),
            pl.BlockSpec((B, H), full2),
            pl.BlockSpec((B, H), full2),
            pl.BlockSpec((3, H, H), full3),
            pl.BlockSpec((3, 1, H), full3),
            pl.BlockSpec((3, 1, H), full3),
            pl.BlockSpec((3, H, H), full3),
            pl.BlockSpec((3, 1, H), full3),
            pl.BlockSpec((3, 1, H), full3),
            pl.BlockSpec((4, H, H), full3),
            pl.BlockSpec((1, H), full2),
        ],
        out_specs=pl.BlockSpec((B, H), full2),
        out_shape=jax.ShapeDtypeStruct((B, H), jnp.float32),
    )(hA, hF, atom_rep[:, -1, :], frag_rep[:, -1, :],
      _split_gates(atom_Wih_b), atom_bih_b.reshape(3, 1, H),
      atom_bhh_b.reshape(3, 1, H),
      _split_gates(frag_Wih_b), frag_bih_b.reshape(3, 1, H),
      frag_bhh_b.reshape(3, 1, H),
      projT, proj_b.reshape(1, H))
    return out


# speedup vs baseline: 1.1501x; 1.1501x over previous
import functools

import jax
import jax.numpy as jnp
from jax.experimental import pallas as pl
from jax.experimental.pallas import tpu as pltpu


def _gru_scan_body(T, x_hbm, wi_ref, wh_ref, bi_ref, bh_ref, hf_ref,
                   xbuf, gr_s, gz_s, gn_s, sem):
    c = pl.program_id(0)
    nc = pl.num_programs(0)

    def copy(chunk, buf, t):
        return pltpu.make_async_copy(
            x_hbm.at[:, chunk * T + t, :], xbuf.at[buf, t], sem.at[buf, t])

    @pl.when(c == 0)
    def _prologue():
        hf_ref[...] = jnp.zeros_like(hf_ref)
        for t in range(T):
            copy(c, 0, t).start()

    @pl.when(c + 1 < nc)
    def _prefetch():
        nxt = c + 1
        for t in range(T):
            copy(nxt, (nxt % 2), t).start()

    buf = c % 2
    for t in range(T):
        copy(c, buf, t).wait()

    x = xbuf[buf]
    dn = (((2,), (0,)), ((), ()))
    f32 = jnp.float32
    gr_s[...] = jax.lax.dot_general(x, wi_ref[0], dn,
                                    preferred_element_type=f32) + bi_ref[0]
    gz_s[...] = jax.lax.dot_general(x, wi_ref[1], dn,
                                    preferred_element_type=f32) + bi_ref[1]
    gn_s[...] = jax.lax.dot_general(x, wi_ref[2], dn,
                                    preferred_element_type=f32) + bi_ref[2]

    whr = wh_ref[0]
    whz = wh_ref[1]
    whn = wh_ref[2]
    bhr = bh_ref[0]
    bhz = bh_ref[1]
    bhn = bh_ref[2]

    def step(t, h):
        ghr = jnp.dot(h, whr, preferred_element_type=f32) + bhr
        ghz = jnp.dot(h, whz, preferred_element_type=f32) + bhz
        ghn = jnp.dot(h, whn, preferred_element_type=f32) + bhn
        r = jax.nn.sigmoid(gr_s[t] + ghr)
        z = jax.nn.sigmoid(gz_s[t] + ghz)
        n = jnp.tanh(gn_s[t] + r * ghn)
        return (1.0 - z) * n + z * h

    h = jax.lax.fori_loop(0, T, step, hf_ref[...])
    hf_ref[...] = h


def _fwd_last_hidden(xs, wi, wh, bi, bh, T):
    B, S, H = xs.shape
    full3 = lambda c: (0, 0, 0)
    return pl.pallas_call(
        functools.partial(_gru_scan_body, T),
        grid=(S // T,),
        in_specs=[
            pl.BlockSpec(memory_space=pl.ANY),
            pl.BlockSpec((3, H, H), full3),
            pl.BlockSpec((3, H, H), full3),
            pl.BlockSpec((3, 1, H), full3),
            pl.BlockSpec((3, 1, H), full3),
        ],
        out_specs=pl.BlockSpec((B, H), lambda c: (0, 0)),
        out_shape=jax.ShapeDtypeStruct((B, H), jnp.float32),
        scratch_shapes=[
            pltpu.VMEM((2, T, B, H), jnp.float32),
            pltpu.VMEM((T, B, H), jnp.float32),
            pltpu.VMEM((T, B, H), jnp.float32),
            pltpu.VMEM((T, B, H), jnp.float32),
            pltpu.SemaphoreType.DMA((2, T)),
        ],
    )(xs, wi, wh, bi, bh)


def _tail_body(ha_ref, hf_ref, xa_ref, xf_ref,
               awb_ref, abib_ref, abhb_ref,
               fwb_ref, fbib_ref, fbhb_ref,
               p_ref, pb_ref, out_ref):
    f32 = jnp.float32

    def back_step(xt, wb_ref, bib_ref, bhb_ref):
        gr = jnp.dot(xt, wb_ref[0], preferred_element_type=f32) + bib_ref[0]
        gz = jnp.dot(xt, wb_ref[1], preferred_element_type=f32) + bib_ref[1]
        gn = jnp.dot(xt, wb_ref[2], preferred_element_type=f32) + bib_ref[2]
        r = jax.nn.sigmoid(gr + bhb_ref[0])
        z = jax.nn.sigmoid(gz + bhb_ref[1])
        n = jnp.tanh(gn + r * bhb_ref[2])
        return (1.0 - z) * n

    bA = back_step(xa_ref[...], awb_ref, abib_ref, abhb_ref)
    bF = back_step(xf_ref[...], fwb_ref, fbib_ref, fbhb_ref)
    acc = jnp.dot(ha_ref[...], p_ref[0], preferred_element_type=f32)
    acc = acc + jnp.dot(bA, p_ref[1], preferred_element_type=f32)
    acc = acc + jnp.dot(hf_ref[...], p_ref[2], preferred_element_type=f32)
    acc = acc + jnp.dot(bF, p_ref[3], preferred_element_type=f32)
    out_ref[...] = acc + pb_ref[...]


def _split_gates(W):
    H = W.shape[1]
    return jnp.swapaxes(W.reshape(3, H, H), 1, 2)


def kernel(atom_rep, frag_rep,
           atom_Wih_f, atom_Whh_f, atom_bih_f, atom_bhh_f,
           atom_Wih_b, atom_Whh_b, atom_bih_b, atom_bhh_b,
           frag_Wih_f, frag_Whh_f, frag_bih_f, frag_bhh_f,
           frag_Wih_b, frag_Whh_b, frag_bih_b, frag_bhh_b,
           proj_W, proj_b):
    B, S_atom, H = atom_rep.shape

    hA = _fwd_last_hidden(atom_rep, _split_gates(atom_Wih_f),
                          _split_gates(atom_Whh_f),
                          atom_bih_f.reshape(3, 1, H),
                          atom_bhh_f.reshape(3, 1, H), 16)
    hF = _fwd_last_hidden(frag_rep, _split_gates(frag_Wih_f),
                          _split_gates(frag_Whh_f),
                          frag_bih_f.reshape(3, 1, H),
                          frag_bhh_f.reshape(3, 1, H), 16)

    projT = proj_W.T.reshape(4, H, H)
    full3 = lambda: (0, 0, 0)
    full2 = lambda: (0, 0)
    out = pl.pallas_call(
        _tail_body,
        in_specs=[
            pl.BlockSpec((B, H), full2),
            pl.BlockSpec((B, H), full2),
            pl.BlockSpec((B, H), full2),
            pl.BlockSpec((B, H), full2),
            pl.BlockSpec((3, H, H), full3),
            pl.BlockSpec((3, 1, H), full3),
            pl.BlockSpec((3, 1, H), full3),
            pl.BlockSpec((3, H, H), full3),
            pl.BlockSpec((3, 1, H), full3),
            pl.BlockSpec((3, 1, H), full3),
            pl.BlockSpec((4, H, H), full3),
            pl.BlockSpec((1, H), full2),
        ],
        out_specs=pl.BlockSpec((B, H), full2),
        out_shape=jax.ShapeDtypeStruct((B, H), jnp.float32),
    )(hA, hF, atom_rep[:, -1, :], frag_rep[:, -1, :],
      _split_gates(atom_Wih_b), atom_bih_b.reshape(3, 1, H),
      atom_bhh_b.reshape(3, 1, H),
      _split_gates(frag_Wih_b), frag_bih_b.reshape(3, 1, H),
      frag_bhh_b.reshape(3, 1, H),
      projT, proj_b.reshape(1, H))
    return out


# speedup vs baseline: 1.5360x; 1.3355x over previous
import functools

import jax
import jax.numpy as jnp
from jax.experimental import pallas as pl
from jax.experimental.pallas import tpu as pltpu


def _gru_scan_body(T, x_ref, wi_ref, wh_ref, bi_ref, bh_ref, hf_ref,
                   gr_s, gz_s, gn_s):
    c = pl.program_id(0)

    @pl.when(c == 0)
    def _init():
        hf_ref[...] = jnp.zeros_like(hf_ref)

    x = x_ref[...]
    dn = (((2,), (0,)), ((), ()))
    f32 = jnp.float32
    gr_s[...] = jax.lax.dot_general(x, wi_ref[0], dn,
                                    preferred_element_type=f32) + bi_ref[0]
    gz_s[...] = jax.lax.dot_general(x, wi_ref[1], dn,
                                    preferred_element_type=f32) + bi_ref[1]
    gn_s[...] = jax.lax.dot_general(x, wi_ref[2], dn,
                                    preferred_element_type=f32) + bi_ref[2]

    whr = wh_ref[0]
    whz = wh_ref[1]
    whn = wh_ref[2]
    bhr = bh_ref[0]
    bhz = bh_ref[1]
    bhn = bh_ref[2]

    def step(t, h):
        ghr = jnp.dot(h, whr, preferred_element_type=f32) + bhr
        ghz = jnp.dot(h, whz, preferred_element_type=f32) + bhz
        ghn = jnp.dot(h, whn, preferred_element_type=f32) + bhn
        r = jax.nn.sigmoid(gr_s[t] + ghr)
        z = jax.nn.sigmoid(gz_s[t] + ghz)
        n = jnp.tanh(gn_s[t] + r * ghn)
        return (1.0 - z) * n + z * h

    h = jax.lax.fori_loop(0, T, step, hf_ref[...], unroll=4)
    hf_ref[...] = h


def _fwd_last_hidden(xs, wi, wh, bi, bh, T):
    S, B, H = xs.shape
    full3 = lambda c: (0, 0, 0)
    return pl.pallas_call(
        functools.partial(_gru_scan_body, T),
        grid=(S // T,),
        in_specs=[
            pl.BlockSpec((T, B, H), lambda c: (c, 0, 0)),
            pl.BlockSpec((3, H, H), full3),
            pl.BlockSpec((3, H, H), full3),
            pl.BlockSpec((3, 1, H), full3),
            pl.BlockSpec((3, 1, H), full3),
        ],
        out_specs=pl.BlockSpec((B, H), lambda c: (0, 0)),
        out_shape=jax.ShapeDtypeStruct((B, H), jnp.float32),
        scratch_shapes=[
            pltpu.VMEM((T, B, H), jnp.float32),
            pltpu.VMEM((T, B, H), jnp.float32),
            pltpu.VMEM((T, B, H), jnp.float32),
        ],
    )(xs, wi, wh, bi, bh)


def _tail_body(ha_ref, hf_ref, xa_ref, xf_ref,
               awb_ref, abib_ref, abhb_ref,
               fwb_ref, fbib_ref, fbhb_ref,
               p_ref, pb_ref, out_ref):
    f32 = jnp.float32

    def back_step(xt, wb_ref, bib_ref, bhb_ref):
        gr = jnp.dot(xt, wb_ref[0], preferred_element_type=f32) + bib_ref[0]
        gz = jnp.dot(xt, wb_ref[1], preferred_element_type=f32) + bib_ref[1]
        gn = jnp.dot(xt, wb_ref[2], preferred_element_type=f32) + bib_ref[2]
        r = jax.nn.sigmoid(gr + bhb_ref[0])
        z = jax.nn.sigmoid(gz + bhb_ref[1])
        n = jnp.tanh(gn + r * bhb_ref[2])
        return (1.0 - z) * n

    bA = back_step(xa_ref[...], awb_ref, abib_ref, abhb_ref)
    bF = back_step(xf_ref[...], fwb_ref, fbib_ref, fbhb_ref)
    acc = jnp.dot(ha_ref[...], p_ref[0], preferred_element_type=f32)
    acc = acc + jnp.dot(bA, p_ref[1], preferred_element_type=f32)
    acc = acc + jnp.dot(hf_ref[...], p_ref[2], preferred_element_type=f32)
    acc = acc + jnp.dot(bF, p_ref[3], preferred_element_type=f32)
    out_ref[...] = acc + pb_ref[...]


def _split_gates(W):
    H = W.shape[1]
    return jnp.swapaxes(W.reshape(3, H, H), 1, 2)


def kernel(atom_rep, frag_rep,
           atom_Wih_f, atom_Whh_f, atom_bih_f, atom_bhh_f,
           atom_Wih_b, atom_Whh_b, atom_bih_b, atom_bhh_b,
           frag_Wih_f, frag_Whh_f, frag_bih_f, frag_bhh_f,
           frag_Wih_b, frag_Whh_b, frag_bih_b, frag_bhh_b,
           proj_W, proj_b):
    B, S_atom, H = atom_rep.shape

    xa = jnp.swapaxes(atom_rep, 0, 1)
    xf = jnp.swapaxes(frag_rep, 0, 1)
    hA = _fwd_last_hidden(xa, _split_gates(atom_Wih_f),
                          _split_gates(atom_Whh_f),
                          atom_bih_f.reshape(3, 1, H),
                          atom_bhh_f.reshape(3, 1, H), 16)
    hF = _fwd_last_hidden(xf, _split_gates(frag_Wih_f),
                          _split_gates(frag_Whh_f),
                          frag_bih_f.reshape(3, 1, H),
                          frag_bhh_f.reshape(3, 1, H), 16)

    projT = proj_W.T.reshape(4, H, H)
    full3 = lambda: (0, 0, 0)
    full2 = lambda: (0, 0)
    out = pl.pallas_call(
        _tail_body,
        in_specs=[
            pl.BlockSpec((B, H), full2),
            pl.BlockSpec((B, H), full2),
            pl.BlockSpec((B, H), full2),
            pl.BlockSpec((B, H), full2),
            pl.BlockSpec((3, H, H), full3),
            pl.BlockSpec((3, 1, H), full3),
            pl.BlockSpec((3, 1, H), full3),
            pl.BlockSpec((3, H, H), full3),
            pl.BlockSpec((3, 1, H), full3),
            pl.BlockSpec((3, 1, H), full3),
            pl.BlockSpec((4, H, H), full3),
            pl.BlockSpec((1, H), full2),
        ],
        out_specs=pl.BlockSpec((B, H), full2),
        out_shape=jax.ShapeDtypeStruct((B, H), jnp.float32),
    )(hA, hF, atom_rep[:, -1, :], frag_rep[:, -1, :],
      _split_gates(atom_Wih_b), atom_bih_b.reshape(3, 1, H),
      atom_bhh_b.reshape(3, 1, H),
      _split_gates(frag_Wih_b), frag_bih_b.reshape(3, 1, H),
      frag_bhh_b.reshape(3, 1, H),
      projT, proj_b.reshape(1, H))
    return out


# speedup vs baseline: 1.5971x; 1.0398x over previous
import functools

import jax
import jax.numpy as jnp
from jax.experimental import pallas as pl
from jax.experimental.pallas import tpu as pltpu


def _gru_scan_body(T, x_ref, wi_ref, wh_ref, bi_ref, bh_ref, hf_ref,
                   gr_s, gz_s, gn_s):
    c = pl.program_id(0)

    @pl.when(c == 0)
    def _init():
        hf_ref[...] = jnp.zeros_like(hf_ref)

    x = x_ref[...]
    dn = (((2,), (0,)), ((), ()))
    f32 = jnp.float32
    gr_s[...] = jax.lax.dot_general(x, wi_ref[0], dn,
                                    preferred_element_type=f32) + bi_ref[0]
    gz_s[...] = jax.lax.dot_general(x, wi_ref[1], dn,
                                    preferred_element_type=f32) + bi_ref[1]
    gn_s[...] = jax.lax.dot_general(x, wi_ref[2], dn,
                                    preferred_element_type=f32) + bi_ref[2]

    whr = wh_ref[0]
    whz = wh_ref[1]
    whn = wh_ref[2]
    bhr = bh_ref[0]
    bhz = bh_ref[1]
    bhn = bh_ref[2]

    def step(t, h):
        ghr = jnp.dot(h, whr, preferred_element_type=f32) + bhr
        ghz = jnp.dot(h, whz, preferred_element_type=f32) + bhz
        ghn = jnp.dot(h, whn, preferred_element_type=f32) + bhn
        r = jax.nn.sigmoid(gr_s[t] + ghr)
        z = jax.nn.sigmoid(gz_s[t] + ghz)
        n = jnp.tanh(gn_s[t] + r * ghn)
        return (1.0 - z) * n + z * h

    h = jax.lax.fori_loop(0, T, step, hf_ref[...], unroll=True)
    hf_ref[...] = h


def _fwd_last_hidden(xs, wi, wh, bi, bh, T):
    S, B, H = xs.shape
    full3 = lambda c: (0, 0, 0)
    return pl.pallas_call(
        functools.partial(_gru_scan_body, T),
        grid=(S // T,),
        in_specs=[
            pl.BlockSpec((T, B, H), lambda c: (c, 0, 0)),
            pl.BlockSpec((3, H, H), full3),
            pl.BlockSpec((3, H, H), full3),
            pl.BlockSpec((3, 1, H), full3),
            pl.BlockSpec((3, 1, H), full3),
        ],
        out_specs=pl.BlockSpec((B, H), lambda c: (0, 0)),
        out_shape=jax.ShapeDtypeStruct((B, H), jnp.float32),
        scratch_shapes=[
            pltpu.VMEM((T, B, H), jnp.float32),
            pltpu.VMEM((T, B, H), jnp.float32),
            pltpu.VMEM((T, B, H), jnp.float32),
        ],
    )(xs, wi, wh, bi, bh)


def _tail_body(ha_ref, hf_ref, xa_ref, xf_ref,
               awb_ref, abib_ref, abhb_ref,
               fwb_ref, fbib_ref, fbhb_ref,
               p_ref, pb_ref, out_ref):
    f32 = jnp.float32

    def back_step(xt, wb_ref, bib_ref, bhb_ref):
        gr = jnp.dot(xt, wb_ref[0], preferred_element_type=f32) + bib_ref[0]
        gz = jnp.dot(xt, wb_ref[1], preferred_element_type=f32) + bib_ref[1]
        gn = jnp.dot(xt, wb_ref[2], preferred_element_type=f32) + bib_ref[2]
        r = jax.nn.sigmoid(gr + bhb_ref[0])
        z = jax.nn.sigmoid(gz + bhb_ref[1])
        n = jnp.tanh(gn + r * bhb_ref[2])
        return (1.0 - z) * n

    bA = back_step(xa_ref[...], awb_ref, abib_ref, abhb_ref)
    bF = back_step(xf_ref[...], fwb_ref, fbib_ref, fbhb_ref)
    acc = jnp.dot(ha_ref[...], p_ref[0], preferred_element_type=f32)
    acc = acc + jnp.dot(bA, p_ref[1], preferred_element_type=f32)
    acc = acc + jnp.dot(hf_ref[...], p_ref[2], preferred_element_type=f32)
    acc = acc + jnp.dot(bF, p_ref[3], preferred_element_type=f32)
    out_ref[...] = acc + pb_ref[...]


def _split_gates(W):
    H = W.shape[1]
    return jnp.swapaxes(W.reshape(3, H, H), 1, 2)


def kernel(atom_rep, frag_rep,
           atom_Wih_f, atom_Whh_f, atom_bih_f, atom_bhh_f,
           atom_Wih_b, atom_Whh_b, atom_bih_b, atom_bhh_b,
           frag_Wih_f, frag_Whh_f, frag_bih_f, frag_bhh_f,
           frag_Wih_b, frag_Whh_b, frag_bih_b, frag_bhh_b,
           proj_W, proj_b):
    B, S_atom, H = atom_rep.shape

    xa = jnp.swapaxes(atom_rep, 0, 1)
    xf = jnp.swapaxes(frag_rep, 0, 1)
    hA = _fwd_last_hidden(xa, _split_gates(atom_Wih_f),
                          _split_gates(atom_Whh_f),
                          atom_bih_f.reshape(3, 1, H),
                          atom_bhh_f.reshape(3, 1, H), 16)
    hF = _fwd_last_hidden(xf, _split_gates(frag_Wih_f),
                          _split_gates(frag_Whh_f),
                          frag_bih_f.reshape(3, 1, H),
                          frag_bhh_f.reshape(3, 1, H), 16)

    projT = proj_W.T.reshape(4, H, H)
    full3 = lambda: (0, 0, 0)
    full2 = lambda: (0, 0)
    out = pl.pallas_call(
        _tail_body,
        in_specs=[
            pl.BlockSpec((B, H), full2),
            pl.BlockSpec((B, H), full2),
            pl.BlockSpec((B, H), full2),
            pl.BlockSpec((B, H), full2),
            pl.BlockSpec((3, H, H), full3),
            pl.BlockSpec((3, 1, H), full3),
            pl.BlockSpec((3, 1, H), full3),
            pl.BlockSpec((3, H, H), full3),
            pl.BlockSpec((3, 1, H), full3),
            pl.BlockSpec((3, 1, H), full3),
            pl.BlockSpec((4, H, H), full3),
            pl.BlockSpec((1, H), full2),
        ],
        out_specs=pl.BlockSpec((B, H), full2),
        out_shape=jax.ShapeDtypeStruct((B, H), jnp.float32),
    )(hA, hF, atom_rep[:, -1, :], frag_rep[:, -1, :],
      _split_gates(atom_Wih_b), atom_bih_b.reshape(3, 1, H),
      atom_bhh_b.reshape(3, 1, H),
      _split_gates(frag_Wih_b), frag_bih_b.reshape(3, 1, H),
      frag_bhh_b.reshape(3, 1, H),
      projT, proj_b.reshape(1, H))
    return out


# speedup vs baseline: 1.6118x; 1.0092x over previous
import functools

import jax
import jax.numpy as jnp
from jax.experimental import pallas as pl
from jax.experimental.pallas import tpu as pltpu


def _gru_scan_body(T, x_ref, wi_ref, wh_ref, bi_ref, bh_ref, hf_ref,
                   gr_s, gz_s, gn_s):
    c = pl.program_id(0)

    @pl.when(c == 0)
    def _init():
        hf_ref[...] = jnp.zeros_like(hf_ref)

    x = x_ref[...]
    dn = (((2,), (0,)), ((), ()))
    f32 = jnp.float32
    gr_s[...] = jax.lax.dot_general(x, wi_ref[0], dn,
                                    preferred_element_type=f32) + bi_ref[0]
    gz_s[...] = jax.lax.dot_general(x, wi_ref[1], dn,
                                    preferred_element_type=f32) + bi_ref[1]
    gn_s[...] = jax.lax.dot_general(x, wi_ref[2], dn,
                                    preferred_element_type=f32) + bi_ref[2]

    whr = wh_ref[0]
    whz = wh_ref[1]
    whn = wh_ref[2]
    bhn = bh_ref[0]

    def step(t, h):
        ghr = jnp.dot(h, whr, preferred_element_type=f32)
        ghz = jnp.dot(h, whz, preferred_element_type=f32)
        ghn = jnp.dot(h, whn, preferred_element_type=f32) + bhn
        r = jax.nn.sigmoid(gr_s[t] + ghr)
        z = jax.nn.sigmoid(gz_s[t] + ghz)
        n = jnp.tanh(gn_s[t] + r * ghn)
        return n + z * (h - n)

    h = jax.lax.fori_loop(0, T, step, hf_ref[...], unroll=True)
    hf_ref[...] = h


def _fwd_last_hidden(xs, wi, wh, bi, bh, T):
    S, B, H = xs.shape
    full3 = lambda c: (0, 0, 0)
    return pl.pallas_call(
        functools.partial(_gru_scan_body, T),
        grid=(S // T,),
        in_specs=[
            pl.BlockSpec((T, B, H), lambda c: (c, 0, 0)),
            pl.BlockSpec((3, H, H), full3),
            pl.BlockSpec((3, H, H), full3),
            pl.BlockSpec((3, 1, H), full3),
            pl.BlockSpec((1, 1, H), lambda c: (0, 0, 0)),
        ],
        out_specs=pl.BlockSpec((B, H), lambda c: (0, 0)),
        out_shape=jax.ShapeDtypeStruct((B, H), jnp.float32),
        scratch_shapes=[
            pltpu.VMEM((T, B, H), jnp.float32),
            pltpu.VMEM((T, B, H), jnp.float32),
            pltpu.VMEM((T, B, H), jnp.float32),
        ],
    )(xs, wi, wh, bi, bh)


def _tail_body(ha_ref, hf_ref, xa_ref, xf_ref,
               awb_ref, abib_ref, abhb_ref,
               fwb_ref, fbib_ref, fbhb_ref,
               p_ref, pb_ref, out_ref):
    f32 = jnp.float32

    def back_step(xt, wb_ref, bib_ref, bhb_ref):
        gr = jnp.dot(xt, wb_ref[0], preferred_element_type=f32) + bib_ref[0]
        gz = jnp.dot(xt, wb_ref[1], preferred_element_type=f32) + bib_ref[1]
        gn = jnp.dot(xt, wb_ref[2], preferred_element_type=f32) + bib_ref[2]
        r = jax.nn.sigmoid(gr + bhb_ref[0])
        z = jax.nn.sigmoid(gz + bhb_ref[1])
        n = jnp.tanh(gn + r * bhb_ref[2])
        return (1.0 - z) * n

    bA = back_step(xa_ref[...], awb_ref, abib_ref, abhb_ref)
    bF = back_step(xf_ref[...], fwb_ref, fbib_ref, fbhb_ref)
    acc = jnp.dot(ha_ref[...], p_ref[0], preferred_element_type=f32)
    acc = acc + jnp.dot(bA, p_ref[1], preferred_element_type=f32)
    acc = acc + jnp.dot(hf_ref[...], p_ref[2], preferred_element_type=f32)
    acc = acc + jnp.dot(bF, p_ref[3], preferred_element_type=f32)
    out_ref[...] = acc + pb_ref[...]


def _split_gates(W):
    H = W.shape[1]
    return jnp.swapaxes(W.reshape(3, H, H), 1, 2)


def kernel(atom_rep, frag_rep,
           atom_Wih_f, atom_Whh_f, atom_bih_f, atom_bhh_f,
           atom_Wih_b, atom_Whh_b, atom_bih_b, atom_bhh_b,
           frag_Wih_f, frag_Whh_f, frag_bih_f, frag_bhh_f,
           frag_Wih_b, frag_Whh_b, frag_bih_b, frag_bhh_b,
           proj_W, proj_b):
    B, S_atom, H = atom_rep.shape

    xa = jnp.swapaxes(atom_rep, 0, 1)
    xf = jnp.swapaxes(frag_rep, 0, 1)

    def fold_biases(bih, bhh):
        bi3 = bih.reshape(3, 1, H)
        bh3 = bhh.reshape(3, 1, H)
        rz_mask = jnp.array([1.0, 1.0, 0.0], jnp.float32).reshape(3, 1, 1)
        return bi3 + bh3 * rz_mask, bh3[2:3]

    a_bi, a_bhn = fold_biases(atom_bih_f, atom_bhh_f)
    f_bi, f_bhn = fold_biases(frag_bih_f, frag_bhh_f)
    hA = _fwd_last_hidden(xa, _split_gates(atom_Wih_f),
                          _split_gates(atom_Whh_f), a_bi, a_bhn, 16)
    hF = _fwd_last_hidden(xf, _split_gates(frag_Wih_f),
                          _split_gates(frag_Whh_f), f_bi, f_bhn, 16)

    projT = proj_W.T.reshape(4, H, H)
    full3 = lambda: (0, 0, 0)
    full2 = lambda: (0, 0)
    out = pl.pallas_call(
        _tail_body,
        in_specs=[
            pl.BlockSpec((B, H), full2),
            pl.BlockSpec((B, H), full2),
            pl.BlockSpec((B, H), full2),
            pl.BlockSpec((B, H), full2),
            pl.BlockSpec((3, H, H), full3),
            pl.BlockSpec((3, 1, H), full3),
            pl.BlockSpec((3, 1, H), full3),
            pl.BlockSpec((3, H, H), full3),
            pl.BlockSpec((3, 1, H), full3),
            pl.BlockSpec((3, 1, H), full3),
            pl.BlockSpec((4, H, H), full3),
            pl.BlockSpec((1, H), full2),
        ],
        out_specs=pl.BlockSpec((B, H), full2),
        out_shape=jax.ShapeDtypeStruct((B, H), jnp.float32),
    )(hA, hF, atom_rep[:, -1, :], frag_rep[:, -1, :],
      _split_gates(atom_Wih_b), atom_bih_b.reshape(3, 1, H),
      atom_bhh_b.reshape(3, 1, H),
      _split_gates(frag_Wih_b), frag_bih_b.reshape(3, 1, H),
      frag_bhh_b.reshape(3, 1, H),
      projT, proj_b.reshape(1, H))
    return out
